# Initial kernel scaffold; baseline (speedup 1.0000x reference)
import functools

import jax
import jax.numpy as jnp
from jax import lax
from jax.experimental import pallas as pl
from jax.experimental.pallas import tpu as pltpu
from jax.experimental.pallas import tpu_sc as plsc

N = 10000
E = 320000
D = 128

NC = 2
NS = 16
NW = NC * NS
B = 128
C = (E + NW * B - 1) // (NW * B)
EP = NW * C * B
NP = 10016
Z = NP // NS
PAD_DST = NP - 1

_f32 = jnp.float32


def _dot_t(a, w):
    return lax.dot_general(a, w, (((1,), (1,)), ((), ())),
                           precision=lax.Precision.HIGHEST,
                           preferred_element_type=_f32)


_RB = 2000


def _tc1_body(x_ref, wl_ref, wr_ref, b_ref, y_ref, r_ref):
    xb = x_ref[...]
    y_ref[...] = _dot_t(xb, wl_ref[...])
    r_ref[...] = _dot_t(xb, wr_ref[...]) + b_ref[...]


def _tc1(x, Wl, Wr, b):
    grid = (N // _RB,)
    return pl.pallas_call(
        _tc1_body,
        grid=grid,
        in_specs=[
            pl.BlockSpec((_RB, D), lambda i: (i, 0)),
            pl.BlockSpec((D, D), lambda i: (0, 0)),
            pl.BlockSpec((D, D), lambda i: (0, 0)),
            pl.BlockSpec((1, D), lambda i: (0, 0)),
        ],
        out_specs=[
            pl.BlockSpec((_RB, D), lambda i: (i, 0)),
            pl.BlockSpec((_RB, D), lambda i: (i, 0)),
        ],
        out_shape=[
            jax.ShapeDtypeStruct((N, D), _f32),
            jax.ShapeDtypeStruct((N, D), _f32),
        ],
    )(x, Wl, Wr, b)


def _tc2_body(p_ref, cnt_ref, r1_ref, wl_ref, wr_ref, b_ref,
              y2_ref, r2_ref, rcp_ref):
    cnt = cnt_ref[0] + cnt_ref[1]
    rcp = 1.0 / jnp.maximum(cnt, 1.0)
    agg = (p_ref[0] + p_ref[1]) * rcp[:, 0:1]
    h = jnp.maximum(agg + r1_ref[...], 0.0)
    y2_ref[...] = _dot_t(h, wl_ref[...])
    r2_ref[...] = _dot_t(h, wr_ref[...]) + b_ref[...]
    rcp_ref[...] = rcp


def _tc2(p, cntp, r1, Wl, Wr, b):
    grid = (N // _RB,)
    return pl.pallas_call(
        _tc2_body,
        grid=grid,
        in_specs=[
            pl.BlockSpec((2, _RB, D), lambda i: (0, i, 0)),
            pl.BlockSpec((2, _RB, 16), lambda i: (0, i, 0)),
            pl.BlockSpec((_RB, D), lambda i: (i, 0)),
            pl.BlockSpec((D, D), lambda i: (0, 0)),
            pl.BlockSpec((D, D), lambda i: (0, 0)),
            pl.BlockSpec((1, D), lambda i: (0, 0)),
        ],
        out_specs=[
            pl.BlockSpec((_RB, D), lambda i: (i, 0)),
            pl.BlockSpec((_RB, D), lambda i: (i, 0)),
            pl.BlockSpec((_RB, 16), lambda i: (i, 0)),
        ],
        out_shape=[
            jax.ShapeDtypeStruct((N, D), _f32),
            jax.ShapeDtypeStruct((N, D), _f32),
            jax.ShapeDtypeStruct((N, 16), _f32),
        ],
    )(p, cntp, r1, Wl, Wr, b)


def _tc3_body(q_ref, rcp_ref, r2_ref, out_ref):
    agg = (q_ref[0] + q_ref[1]) * rcp_ref[:, 0:1]
    out_ref[...] = agg + r2_ref[...]


def _tc3(q, rcp, r2):
    grid = (N // _RB,)
    return pl.pallas_call(
        _tc3_body,
        grid=grid,
        in_specs=[
            pl.BlockSpec((2, _RB, D), lambda i: (0, i, 0)),
            pl.BlockSpec((_RB, 16), lambda i: (i, 0)),
            pl.BlockSpec((_RB, D), lambda i: (i, 0)),
        ],
        out_specs=pl.BlockSpec((_RB, D), lambda i: (i, 0)),
        out_shape=jax.ShapeDtypeStruct((N, D), _f32),
    )(q, rcp, r2)


_MESH = plsc.VectorSubcoreMesh(core_axis_name="c", subcore_axis_name="s")


def _sc_edge_loop(y_hbm, src_v, dst_v, g0, acc_sh, wid,
                  srcT_hbm, dstT_hbm, extra_chunk=None):
    pltpu.sync_copy(srcT_hbm.at[wid], src_v)
    pltpu.sync_copy(dstT_hbm.at[wid], dst_v)
    plsc.subcore_barrier()

    def chunk(i, carry):
        pltpu.sync_copy(y_hbm.at[src_v.at[i]], g0)
        pltpu.sync_copy(g0, acc_sh.at[dst_v.at[i]], add=True)
        if extra_chunk is not None:
            extra_chunk(i)
        return carry

    lax.fori_loop(0, C, chunk, 0)
    plsc.subcore_barrier()


def _sc_agg_counts_body(y_hbm, srcT_hbm, dstT_hbm, z128_hbm, z16_hbm,
                        acc_out, cnt_out,
                        src_v, dst_v, g0, g1, ones_v, acc_sh, cnt_sh):
    c = lax.axis_index("c")
    s = lax.axis_index("s")
    wid = c * NS + s

    pltpu.sync_copy(z128_hbm, acc_sh.at[pl.ds(s * Z, Z)])
    pltpu.sync_copy(z16_hbm, cnt_sh.at[pl.ds(s * Z, Z)])

    def fill_ones(i, carry):
        ones_v[i, :] = jnp.ones((16,), _f32)
        return carry
    lax.fori_loop(0, B, fill_ones, 0)

    def extra(i):
        pltpu.sync_copy(ones_v, cnt_sh.at[dst_v.at[i]], add=True)

    _sc_edge_loop(y_hbm, src_v, dst_v, g0, acc_sh, wid,
                  srcT_hbm, dstT_hbm, extra_chunk=extra)

    pltpu.sync_copy(acc_sh.at[pl.ds(s * Z, Z)], acc_out.at[c, pl.ds(s * Z, Z)])
    pltpu.sync_copy(cnt_sh.at[pl.ds(s * Z, Z)], cnt_out.at[c, pl.ds(s * Z, Z)])


_sc_agg_counts = functools.partial(
    pl.kernel,
    out_type=(jax.ShapeDtypeStruct((NC, NP, D), _f32),
              jax.ShapeDtypeStruct((NC, NP, 16), _f32)),
    mesh=_MESH,
    scratch_types=[
        pltpu.VMEM((C, B), jnp.int32),
        pltpu.VMEM((C, B), jnp.int32),
        pltpu.VMEM((B, D), _f32),
        pltpu.VMEM((B, D), _f32),
        pltpu.VMEM((B, 16), _f32),
        pltpu.VMEM_SHARED((NP, D), _f32),
        pltpu.VMEM_SHARED((NP, 16), _f32),
    ],
)(_sc_agg_counts_body)


def _sc_agg_body(y_hbm, srcT_hbm, dstT_hbm, z128_hbm,
                 acc_out,
                 src_v, dst_v, g0, g1, acc_sh):
    c = lax.axis_index("c")
    s = lax.axis_index("s")
    wid = c * NS + s

    pltpu.sync_copy(z128_hbm, acc_sh.at[pl.ds(s * Z, Z)])

    _sc_edge_loop(y_hbm, src_v, dst_v, g0, acc_sh, wid,
                  srcT_hbm, dstT_hbm)

    pltpu.sync_copy(acc_sh.at[pl.ds(s * Z, Z)], acc_out.at[c, pl.ds(s * Z, Z)])


_sc_agg = functools.partial(
    pl.kernel,
    out_type=jax.ShapeDtypeStruct((NC, NP, D), _f32),
    mesh=_MESH,
    scratch_types=[
        pltpu.VMEM((C, B), jnp.int32),
        pltpu.VMEM((C, B), jnp.int32),
        pltpu.VMEM((B, D), _f32),
        pltpu.VMEM((B, D), _f32),
        pltpu.VMEM_SHARED((NP, D), _f32),
    ],
)(_sc_agg_body)


@jax.jit
def kernel(x, edge_index, W1l, b1l, W1r, W2l, b2l, W2r):
    src = edge_index[0]
    dst = edge_index[1]
    pad = EP - E
    srcT = jnp.concatenate([src, jnp.zeros((pad,), jnp.int32)]).reshape(NW, C, B)
    dstT = jnp.concatenate([dst, jnp.full((pad,), PAD_DST, jnp.int32)]).reshape(NW, C, B)
    z128 = jnp.zeros((Z, D), _f32)
    z16 = jnp.zeros((Z, 16), _f32)
    b1 = b1l.reshape(1, D)
    b2 = b2l.reshape(1, D)

    y1, r1 = _tc1(x, W1l, W1r, b1)
    p, cntp = _sc_agg_counts(y1, srcT, dstT, z128, z16)
    y2, r2, rcp = _tc2(p, cntp, r1, W2l, W2r, b2)
    q = _sc_agg(y2, srcT, dstT, z128)
    return _tc3(q, rcp, r2)


# speedup vs baseline: 5.1204x; 5.1204x over previous
import functools

import jax
import jax.numpy as jnp
from jax import lax
from jax.experimental import pallas as pl
from jax.experimental.pallas import tpu as pltpu
from jax.experimental.pallas import tpu_sc as plsc

N = 10000
E = 320000
D = 128

NC = 2
NS = 16
NW = NC * NS
B = 128
C = (E + NW * B - 1) // (NW * B)
EP = NW * C * B
NP = 10112
Z = NP // NS
PAD_DST = NP - 1

_f32 = jnp.float32


def _dot_t(a, w):
    return lax.dot_general(a, w, (((1,), (1,)), ((), ())),
                           precision=lax.Precision.HIGHEST,
                           preferred_element_type=_f32)


_RB = 2000


def _tc1_body(x_ref, wl_ref, wr_ref, b_ref, y_ref, r_ref):
    xb = x_ref[...]
    y_ref[...] = _dot_t(xb, wl_ref[...])
    r_ref[...] = _dot_t(xb, wr_ref[...]) + b_ref[...]


def _tc1(x, Wl, Wr, b):
    grid = (N // _RB,)
    return pl.pallas_call(
        _tc1_body,
        grid=grid,
        in_specs=[
            pl.BlockSpec((_RB, D), lambda i: (i, 0)),
            pl.BlockSpec((D, D), lambda i: (0, 0)),
            pl.BlockSpec((D, D), lambda i: (0, 0)),
            pl.BlockSpec((1, D), lambda i: (0, 0)),
        ],
        out_specs=[
            pl.BlockSpec((_RB, D), lambda i: (i, 0)),
            pl.BlockSpec((_RB, D), lambda i: (i, 0)),
        ],
        out_shape=[
            jax.ShapeDtypeStruct((N, D), _f32),
            jax.ShapeDtypeStruct((N, D), _f32),
        ],
    )(x, Wl, Wr, b)


def _tc2_body(p_ref, cnt_ref, r1_ref, wl_ref, wr_ref, b_ref,
              y2_ref, r2_ref, rcp_ref):
    cnt = cnt_ref[0] + cnt_ref[1]
    rcp = 1.0 / jnp.maximum(cnt, 1.0)
    agg = (p_ref[0] + p_ref[1]) * rcp[:, 0:1]
    h = jnp.maximum(agg + r1_ref[...], 0.0)
    y2_ref[...] = _dot_t(h, wl_ref[...])
    r2_ref[...] = _dot_t(h, wr_ref[...]) + b_ref[...]
    rcp_ref[...] = rcp


def _tc2(p, cntp, r1, Wl, Wr, b):
    grid = (N // _RB,)
    return pl.pallas_call(
        _tc2_body,
        grid=grid,
        in_specs=[
            pl.BlockSpec((2, _RB, D), lambda i: (0, i, 0)),
            pl.BlockSpec((2, _RB, 16), lambda i: (0, i, 0)),
            pl.BlockSpec((_RB, D), lambda i: (i, 0)),
            pl.BlockSpec((D, D), lambda i: (0, 0)),
            pl.BlockSpec((D, D), lambda i: (0, 0)),
            pl.BlockSpec((1, D), lambda i: (0, 0)),
        ],
        out_specs=[
            pl.BlockSpec((_RB, D), lambda i: (i, 0)),
            pl.BlockSpec((_RB, D), lambda i: (i, 0)),
            pl.BlockSpec((_RB, 16), lambda i: (i, 0)),
        ],
        out_shape=[
            jax.ShapeDtypeStruct((N, D), _f32),
            jax.ShapeDtypeStruct((N, D), _f32),
            jax.ShapeDtypeStruct((N, 16), _f32),
        ],
    )(p, cntp, r1, Wl, Wr, b)


def _tc3_body(q_ref, rcp_ref, r2_ref, out_ref):
    agg = (q_ref[0] + q_ref[1]) * rcp_ref[:, 0:1]
    out_ref[...] = agg + r2_ref[...]


def _tc3(q, rcp, r2):
    grid = (N // _RB,)
    return pl.pallas_call(
        _tc3_body,
        grid=grid,
        in_specs=[
            pl.BlockSpec((2, _RB, D), lambda i: (0, i, 0)),
            pl.BlockSpec((_RB, 16), lambda i: (i, 0)),
            pl.BlockSpec((_RB, D), lambda i: (i, 0)),
        ],
        out_specs=pl.BlockSpec((_RB, D), lambda i: (i, 0)),
        out_shape=jax.ShapeDtypeStruct((N, D), _f32),
    )(q, rcp, r2)


_MESH = plsc.VectorSubcoreMesh(core_axis_name="c", subcore_axis_name="s")


def _sc_edge_loop(y_hbm, src_v, dst_v, g0, acc_sh, wid,
                  srcT_hbm, dstT_hbm, extra_chunk=None):
    pltpu.sync_copy(srcT_hbm.at[wid], src_v)
    pltpu.sync_copy(dstT_hbm.at[wid], dst_v)
    plsc.subcore_barrier()

    def chunk(i, carry):
        pltpu.sync_copy(y_hbm.at[src_v.at[i]], g0)
        pltpu.sync_copy(g0, acc_sh.at[dst_v.at[i]], add=True)
        if extra_chunk is not None:
            extra_chunk(i)
        return carry

    lax.fori_loop(0, C, chunk, 0)
    plsc.subcore_barrier()


def _sc_counts_body(dstT_hbm, z16_hbm, ones_hbm,
                    cnt_out,
                    dst_v, ones_v, cnt_sh):
    c = lax.axis_index("c")
    s = lax.axis_index("s")
    wid = c * NS + s

    pltpu.sync_copy(z16_hbm, cnt_sh.at[pl.ds(s * Z, Z)])
    pltpu.sync_copy(dstT_hbm.at[wid], dst_v)
    pltpu.sync_copy(ones_hbm, ones_v)
    plsc.subcore_barrier()

    def chunk(i, carry):
        pltpu.sync_copy(ones_v, cnt_sh.at[dst_v.at[i]], add=True)
        return carry

    lax.fori_loop(0, C, chunk, 0)
    plsc.subcore_barrier()
    pltpu.sync_copy(cnt_sh.at[pl.ds(s * Z, Z)], cnt_out.at[c, pl.ds(s * Z, Z)])


_sc_counts = functools.partial(
    pl.kernel,
    out_type=jax.ShapeDtypeStruct((NC, NP, 16), _f32),
    mesh=_MESH,
    scratch_types=[
        pltpu.VMEM((C, B), jnp.int32),
        pltpu.VMEM((B, 16), _f32),
        pltpu.VMEM_SHARED((NP, 16), _f32),
    ],
    compiler_params=pltpu.CompilerParams(use_tc_tiling_on_sc=False),
)(_sc_counts_body)


def _sc_agg_body(y_hbm, srcT_hbm, dstT_hbm, z128_hbm,
                 acc_out,
                 src_v, dst_v, g0, g1, acc_sh):
    c = lax.axis_index("c")
    s = lax.axis_index("s")
    wid = c * NS + s

    pltpu.sync_copy(z128_hbm, acc_sh.at[pl.ds(s * Z, Z)])

    _sc_edge_loop(y_hbm, src_v, dst_v, g0, acc_sh, wid,
                  srcT_hbm, dstT_hbm)

    pltpu.sync_copy(acc_sh.at[pl.ds(s * Z, Z)], acc_out.at[c, pl.ds(s * Z, Z)])


_sc_agg = functools.partial(
    pl.kernel,
    out_type=jax.ShapeDtypeStruct((NC, NP, D), _f32),
    mesh=_MESH,
    scratch_types=[
        pltpu.VMEM((C, B), jnp.int32),
        pltpu.VMEM((C, B), jnp.int32),
        pltpu.VMEM((B, D), _f32),
        pltpu.VMEM((B, D), _f32),
        pltpu.VMEM_SHARED((NP, D), _f32),
    ],
)(_sc_agg_body)


@jax.jit
def kernel(x, edge_index, W1l, b1l, W1r, W2l, b2l, W2r):
    src = edge_index[0]
    dst = edge_index[1]
    pad = EP - E
    srcT = jnp.concatenate([src, jnp.zeros((pad,), jnp.int32)]).reshape(NW, C, B)
    dstT = jnp.concatenate([dst, jnp.full((pad,), PAD_DST, jnp.int32)]).reshape(NW, C, B)
    z128 = jnp.zeros((Z, D), _f32)
    z16 = jnp.zeros((Z, 16), _f32)
    b1 = b1l.reshape(1, D)
    b2 = b2l.reshape(1, D)

    ones = jnp.ones((B, 16), _f32)
    cntp = _sc_counts(dstT, z16, ones)
    y1, r1 = _tc1(x, W1l, W1r, b1)
    p = _sc_agg(y1, srcT, dstT, z128)
    y2, r2, rcp = _tc2(p, cntp, r1, W2l, W2r, b2)
    q = _sc_agg(y2, srcT, dstT, z128)
    return _tc3(q, rcp, r2)
